# Initial kernel scaffold; baseline (speedup 1.0000x reference)
import jax
import jax.numpy as jnp
from jax.experimental import pallas as pl


def kernel(x, edge_index, edge_weight, W, b):
    raise NotImplementedError("write your pallas kernel here")


# speedup vs baseline: 9.8564x; 9.8564x over previous
import functools

import jax
import jax.numpy as jnp
from jax import lax
from jax.experimental import pallas as pl
from jax.experimental.pallas import tpu as pltpu
from jax.experimental.pallas import tpu_sc as plsc

N = 10000
E = 320000
D = 128
C = 64

NC = 2
NS = 16
LANES = 16
NW = NC * NS

CHUNK = 128
N_PAD = 10240
N_CHUNKS = -(-E // (NW * CHUNK))
E_PAD = NW * N_CHUNKS * CHUNK
STRIPE = N_PAD // NS
NSUB = STRIPE // CHUNK


def _sc_mesh():
    return plsc.VectorSubcoreMesh(core_axis_name="c", subcore_axis_name="s")


@functools.partial(
    pl.kernel,
    out_type=jax.ShapeDtypeStruct((NC, N_PAD, LANES), jnp.float32),
    mesh=_sc_mesh(),
    scratch_types=[
        pltpu.VMEM((N_CHUNKS, CHUNK), jnp.int32),
        pltpu.VMEM((CHUNK, LANES), jnp.float32),
        pltpu.VMEM_SHARED((N_PAD, LANES), jnp.float32),
    ],
    compiler_params=pltpu.CompilerParams(use_tc_tiling_on_sc=False),
)
def _deg_kernel(col_hbm, ew16_hbm, out_hbm, col_v, rows_v, acc):
    cid = lax.axis_index("c")
    sid = lax.axis_index("s")
    wid = sid * NC + cid

    def zrow(i, _):
        rows_v[i, :] = jnp.zeros((LANES,), jnp.float32)
        return 0
    lax.fori_loop(0, CHUNK, zrow, 0)

    def zstripe(s, _):
        pltpu.sync_copy(rows_v, acc.at[pl.ds(sid * STRIPE + s * CHUNK, CHUNK)])
        return 0
    lax.fori_loop(0, NSUB, zstripe, 0)
    plsc.subcore_barrier()

    pltpu.sync_copy(col_hbm.at[wid], col_v)

    def chunk_body(ch, _):
        pltpu.sync_copy(ew16_hbm.at[wid, ch], rows_v)
        pltpu.sync_copy(rows_v, acc.at[col_v.at[ch]], add=True)
        return 0
    lax.fori_loop(0, N_CHUNKS, chunk_body, 0)

    plsc.subcore_barrier()

    def drain(s, _):
        off = sid * STRIPE + s * CHUNK
        pltpu.sync_copy(acc.at[pl.ds(off, CHUNK)], rows_v)
        pltpu.sync_copy(rows_v, out_hbm.at[cid, pl.ds(off, CHUNK)])
        return 0
    lax.fori_loop(0, NSUB, drain, 0)


@functools.partial(
    pl.kernel,
    out_type=jax.ShapeDtypeStruct((NC, N_PAD, C), jnp.float32),
    mesh=_sc_mesh(),
    scratch_types=[
        pltpu.VMEM((N_CHUNKS, CHUNK), jnp.int32),
        pltpu.VMEM((N_CHUNKS, CHUNK), jnp.int32),
        pltpu.VMEM((CHUNK, LANES), jnp.float32),
        pltpu.VMEM((CHUNK, C), jnp.float32),
        pltpu.VMEM_SHARED((N_PAD, C), jnp.float32),
        pltpu.SemaphoreType.DMA,
    ],
    compiler_params=pltpu.CompilerParams(use_tc_tiling_on_sc=False),
)
def _prop_kernel(u_hbm, row_hbm, col_hbm, ew16_hbm, out_hbm,
                 row_v, col_v, ew_v, rows_v, acc, sem):
    cid = lax.axis_index("c")
    sid = lax.axis_index("s")
    wid = sid * NC + cid

    nz = CHUNK * C // LANES

    def zrow(i, _):
        rows_v[i // (C // LANES), pl.ds((i % (C // LANES)) * LANES, LANES)] = (
            jnp.zeros((LANES,), jnp.float32))
        return 0
    lax.fori_loop(0, nz, zrow, 0)

    def zstripe(s, _):
        pltpu.sync_copy(rows_v, acc.at[pl.ds(sid * STRIPE + s * CHUNK, CHUNK)])
        return 0
    lax.fori_loop(0, NSUB, zstripe, 0)
    plsc.subcore_barrier()

    pltpu.sync_copy(row_hbm.at[wid], row_v)
    pltpu.sync_copy(col_hbm.at[wid], col_v)

    def chunk_body(ch, _):
        pltpu.sync_copy(ew16_hbm.at[wid, ch], ew_v)
        pltpu.async_copy(u_hbm.at[row_v.at[ch]], rows_v, sem).wait()

        def edge_body(k, _):
            w = ew_v[k, :]
            for j in range(C // LANES):
                sl = pl.ds(j * LANES, LANES)
                rows_v[k, sl] = rows_v[k, sl] * w
            return 0
        lax.fori_loop(0, CHUNK, edge_body, 0)

        pltpu.sync_copy(rows_v, acc.at[col_v.at[ch]], add=True)
        return 0
    lax.fori_loop(0, N_CHUNKS, chunk_body, 0)

    plsc.subcore_barrier()

    def drain(s, _):
        off = sid * STRIPE + s * CHUNK
        pltpu.sync_copy(acc.at[pl.ds(off, CHUNK)], rows_v)
        pltpu.sync_copy(rows_v, out_hbm.at[cid, pl.ds(off, CHUNK)])
        return 0
    lax.fori_loop(0, NSUB, drain, 0)


BLK = 256


def _tc_deg_norm(dp0, dp1):
    def body(a_ref, b_ref, dis_ref, ideg_ref):
        deg = a_ref[...] + b_ref[...] + 1.0
        dis_ref[...] = lax.rsqrt(deg)
        ideg_ref[...] = 1.0 / deg

    return pl.pallas_call(
        body,
        grid=(N_PAD // BLK,),
        in_specs=[
            pl.BlockSpec((BLK, LANES), lambda i: (i, 0)),
            pl.BlockSpec((BLK, LANES), lambda i: (i, 0)),
        ],
        out_specs=[
            pl.BlockSpec((BLK, LANES), lambda i: (i, 0)),
            pl.BlockSpec((BLK, LANES), lambda i: (i, 0)),
        ],
        out_shape=[
            jax.ShapeDtypeStruct((N_PAD, LANES), jnp.float32),
            jax.ShapeDtypeStruct((N_PAD, LANES), jnp.float32),
        ],
    )(dp0, dp1)


def _tc_project(x_pad, wt, dis):
    def body(x_ref, w_ref, s_ref, o_ref):
        y = jnp.dot(x_ref[...], w_ref[...], preferred_element_type=jnp.float32)
        o_ref[...] = y * s_ref[:, 0:1]

    return pl.pallas_call(
        body,
        grid=(N_PAD // BLK,),
        in_specs=[
            pl.BlockSpec((BLK, D), lambda i: (i, 0)),
            pl.BlockSpec((D, C), lambda i: (0, 0)),
            pl.BlockSpec((BLK, LANES), lambda i: (i, 0)),
        ],
        out_specs=pl.BlockSpec((BLK, C), lambda i: (i, 0)),
        out_shape=jax.ShapeDtypeStruct((N_PAD, C), jnp.float32),
    )(x_pad, wt, dis)


def _tc_combine(u, p0, p1, s):
    def body(u_ref, a_ref, b_ref, s_ref, o_ref):
        o_ref[...] = (u_ref[...] + a_ref[...] + b_ref[...]) * s_ref[:, 0:1]

    return pl.pallas_call(
        body,
        grid=(N_PAD // BLK,),
        in_specs=[
            pl.BlockSpec((BLK, C), lambda i: (i, 0)),
            pl.BlockSpec((BLK, C), lambda i: (i, 0)),
            pl.BlockSpec((BLK, C), lambda i: (i, 0)),
            pl.BlockSpec((BLK, LANES), lambda i: (i, 0)),
        ],
        out_specs=pl.BlockSpec((BLK, C), lambda i: (i, 0)),
        out_shape=jax.ShapeDtypeStruct((N_PAD, C), jnp.float32),
    )(u, p0, p1, s)


def _tc_final(u, p0, p1, s, b2d):
    def body(u_ref, a_ref, b_ref, s_ref, bias_ref, o_ref):
        o_ref[...] = (u_ref[...] + a_ref[...] + b_ref[...]) * s_ref[:, 0:1] + bias_ref[...]

    return pl.pallas_call(
        body,
        grid=(N_PAD // BLK,),
        in_specs=[
            pl.BlockSpec((BLK, C), lambda i: (i, 0)),
            pl.BlockSpec((BLK, C), lambda i: (i, 0)),
            pl.BlockSpec((BLK, C), lambda i: (i, 0)),
            pl.BlockSpec((BLK, LANES), lambda i: (i, 0)),
            pl.BlockSpec((1, C), lambda i: (0, 0)),
        ],
        out_specs=pl.BlockSpec((BLK, C), lambda i: (i, 0)),
        out_shape=jax.ShapeDtypeStruct((N_PAD, C), jnp.float32),
    )(u, p0, p1, s, b2d)


@jax.jit
def kernel(x, edge_index, edge_weight, W, b):
    row = edge_index[0].astype(jnp.int32)
    col = edge_index[1].astype(jnp.int32)
    ew = edge_weight.astype(jnp.float32)

    pad = E_PAD - E
    row_r = jnp.pad(row, (0, pad)).reshape(NW, N_CHUNKS, CHUNK)
    col_r = jnp.pad(col, (0, pad)).reshape(NW, N_CHUNKS, CHUNK)
    ew16_r = jnp.broadcast_to(
        jnp.pad(ew, (0, pad)).reshape(NW, N_CHUNKS, CHUNK, 1),
        (NW, N_CHUNKS, CHUNK, LANES)).reshape(NW, N_CHUNKS, CHUNK, LANES)

    x_pad = jnp.pad(x, ((0, N_PAD - N), (0, 0)))
    wt = W.T
    b2d = b.reshape(1, C)

    dp = _deg_kernel(col_r, ew16_r)
    dis, ideg = _tc_deg_norm(dp[0], dp[1])
    u0 = _tc_project(x_pad, wt, dis)
    p = _prop_kernel(u0, row_r, col_r, ew16_r)
    u1 = _tc_combine(u0, p[0], p[1], ideg)
    q = _prop_kernel(u1, row_r, col_r, ew16_r)
    out = _tc_final(u1, q[0], q[1], dis, b2d)
    return out[:N]


# speedup vs baseline: 12.2393x; 1.2418x over previous
import functools

import jax
import jax.numpy as jnp
from jax import lax
from jax.experimental import pallas as pl
from jax.experimental.pallas import tpu as pltpu
from jax.experimental.pallas import tpu_sc as plsc

N = 10000
E = 320000
D = 128
C = 64

NC = 2
NS = 16
LANES = 16
NW = NC * NS

CHUNK = 128
N_PAD = 10240
N_CHUNKS = 80
E_PAD = NW * N_CHUNKS * CHUNK
NBANK = 2
NSTEPS = N_CHUNKS // NBANK
NITER = NSTEPS // 2
DBANK = 8
DSTEPS = N_CHUNKS // DBANK
DITER = DSTEPS // 2
STRIPE = N_PAD // NS
NSUB = STRIPE // CHUNK


def _sc_mesh():
    return plsc.VectorSubcoreMesh(core_axis_name="c", subcore_axis_name="s")


@functools.partial(
    pl.kernel,
    out_type=jax.ShapeDtypeStruct((NC, N_PAD, LANES), jnp.float32),
    mesh=_sc_mesh(),
    scratch_types=[
        pltpu.VMEM((N_CHUNKS, CHUNK), jnp.int32),
        pltpu.VMEM((DBANK, CHUNK, LANES), jnp.float32),
        pltpu.VMEM((DBANK, CHUNK, LANES), jnp.float32),
        pltpu.VMEM((CHUNK, LANES), jnp.float32),
        pltpu.VMEM_SHARED((N_PAD, LANES), jnp.float32),
        pltpu.SemaphoreType.DMA,
        pltpu.SemaphoreType.DMA,
        pltpu.SemaphoreType.DMA,
    ],
    compiler_params=pltpu.CompilerParams(use_tc_tiling_on_sc=False),
)
def _deg_kernel(col_hbm, ew16_hbm, out_hbm, col_v, ewa, ewb, zbuf, acc,
                esema, esemb, ssem):
    cid = lax.axis_index("c")
    sid = lax.axis_index("s")
    wid = sid * NC + cid

    pltpu.async_copy(col_hbm.at[wid], col_v, ssem)

    def zrow(i, _):
        zbuf[i, :] = jnp.zeros((LANES,), jnp.float32)
        return 0
    lax.fori_loop(0, CHUNK, zrow, 0)

    def zstripe(s, _):
        pltpu.sync_copy(zbuf, acc.at[pl.ds(sid * STRIPE + s * CHUNK, CHUNK)])
        return 0
    lax.fori_loop(0, NSUB, zstripe, 0)
    pltpu.make_async_copy(col_hbm.at[wid], col_v, ssem).wait()
    plsc.subcore_barrier()

    pltpu.async_copy(ew16_hbm.at[wid, pl.ds(0, DBANK)], ewa, esema)
    pltpu.async_copy(ew16_hbm.at[wid, pl.ds(DBANK, DBANK)], ewb, esemb)

    def iter_body(t, _):
        for bank, ew_v, esem in ((0, ewa, esema), (1, ewb, esemb)):
            s = 2 * t + bank
            pltpu.make_async_copy(
                ew16_hbm.at[wid, pl.ds(0, DBANK)], ew_v, esem).wait()
            for b in range(DBANK):
                ch = s * DBANK + b
                pltpu.async_copy(ew_v.at[b], acc.at[col_v.at[ch]], ssem,
                                 add=True)
            for b in range(DBANK):
                pltpu.make_async_copy(
                    ew_v.at[b], acc.at[col_v.at[0]], ssem).wait()

            @pl.when(t < DITER - 1)
            def _():
                pltpu.async_copy(
                    ew16_hbm.at[wid, pl.ds((s + 2) * DBANK, DBANK)], ew_v, esem)
        return 0
    lax.fori_loop(0, DITER, iter_body, 0)

    plsc.subcore_barrier()

    def drain(s, _):
        off = sid * STRIPE + s * CHUNK
        pltpu.sync_copy(acc.at[pl.ds(off, CHUNK)], zbuf)
        pltpu.sync_copy(zbuf, out_hbm.at[cid, pl.ds(off, CHUNK)])
        return 0
    lax.fori_loop(0, NSUB, drain, 0)


@functools.partial(
    pl.kernel,
    out_type=jax.ShapeDtypeStruct((NC, N_PAD, C), jnp.float32),
    mesh=_sc_mesh(),
    scratch_types=[
        pltpu.VMEM((N_CHUNKS, CHUNK), jnp.int32),
        pltpu.VMEM((N_CHUNKS, CHUNK), jnp.int32),
        pltpu.VMEM((NBANK, CHUNK, LANES), jnp.float32),
        pltpu.VMEM((NBANK, CHUNK, LANES), jnp.float32),
        pltpu.VMEM((NBANK, CHUNK, C), jnp.float32),
        pltpu.VMEM((NBANK, CHUNK, C), jnp.float32),
        pltpu.VMEM_SHARED((N_PAD, C), jnp.float32),
        pltpu.SemaphoreType.DMA,
        pltpu.SemaphoreType.DMA,
        pltpu.SemaphoreType.DMA,
        pltpu.SemaphoreType.DMA,
        pltpu.SemaphoreType.DMA,
    ],
    compiler_params=pltpu.CompilerParams(use_tc_tiling_on_sc=False),
)
def _prop_kernel(u_hbm, row_hbm, col_hbm, ew16_hbm, out_hbm,
                 row_v, col_v, ewa, ewb, bufa, bufb, acc,
                 gsema, gsemb, esema, esemb, ssem):
    cid = lax.axis_index("c")
    sid = lax.axis_index("s")
    wid = sid * NC + cid

    pltpu.async_copy(row_hbm.at[wid], row_v, ssem)
    pltpu.async_copy(col_hbm.at[wid], col_v, ssem)

    zbuf = bufa.at[0]
    nz = CHUNK * C // LANES

    def zrow(i, _):
        zbuf[i // (C // LANES), pl.ds((i % (C // LANES)) * LANES, LANES)] = (
            jnp.zeros((LANES,), jnp.float32))
        return 0
    lax.fori_loop(0, nz, zrow, 0)

    def zstripe(s, _):
        pltpu.sync_copy(zbuf, acc.at[pl.ds(sid * STRIPE + s * CHUNK, CHUNK)])
        return 0
    lax.fori_loop(0, NSUB, zstripe, 0)
    pltpu.make_async_copy(row_hbm.at[wid], row_v, ssem).wait()
    pltpu.make_async_copy(col_hbm.at[wid], col_v, ssem).wait()
    plsc.subcore_barrier()

    pltpu.async_copy(ew16_hbm.at[wid, pl.ds(0, NBANK)], ewa, esema)
    pltpu.async_copy(ew16_hbm.at[wid, pl.ds(NBANK, NBANK)], ewb, esemb)
    for b in range(NBANK):
        pltpu.async_copy(u_hbm.at[row_v.at[b]], bufa.at[b], gsema)
    for b in range(NBANK):
        pltpu.async_copy(u_hbm.at[row_v.at[NBANK + b]], bufb.at[b], gsemb)

    def iter_body(t, _):
        for bank, ew_v, buf, gsem, esem in (
                (0, ewa, bufa, gsema, esema), (1, ewb, bufb, gsemb, esemb)):
            s = 2 * t + bank
            pltpu.make_async_copy(
                ew16_hbm.at[wid, pl.ds(0, NBANK)], ew_v, esem).wait()
            for b in range(NBANK):
                ch = s * NBANK + b
                pltpu.make_async_copy(
                    u_hbm.at[row_v.at[0]], buf.at[b], gsem).wait()

                def grp(g, _):
                    for t4 in range(4):
                        k = g * 4 + t4
                        w = ew_v[b, k, :]
                        for j in range(C // LANES):
                            sl = pl.ds(j * LANES, LANES)
                            buf[b, k, sl] = buf[b, k, sl] * w
                    return 0
                lax.fori_loop(0, CHUNK // 4, grp, 0)
                pltpu.async_copy(buf.at[b], acc.at[col_v.at[ch]], ssem,
                                 add=True)
            for b in range(NBANK):
                pltpu.make_async_copy(
                    buf.at[b], acc.at[col_v.at[0]], ssem).wait()

            @pl.when(t < NITER - 1)
            def _():
                pltpu.async_copy(
                    ew16_hbm.at[wid, pl.ds((s + 2) * NBANK, NBANK)], ew_v, esem)
                for b in range(NBANK):
                    ch2 = (s + 2) * NBANK + b
                    pltpu.async_copy(u_hbm.at[row_v.at[ch2]], buf.at[b], gsem)
        return 0
    lax.fori_loop(0, NITER, iter_body, 0)

    plsc.subcore_barrier()

    dbuf = bufa.at[0]

    def drain(s, _):
        off = sid * STRIPE + s * CHUNK
        pltpu.sync_copy(acc.at[pl.ds(off, CHUNK)], dbuf)
        pltpu.sync_copy(dbuf, out_hbm.at[cid, pl.ds(off, CHUNK)])
        return 0
    lax.fori_loop(0, NSUB, drain, 0)


BLK = 256


def _tc_deg_norm(dp0, dp1):
    def body(a_ref, b_ref, dis_ref, ideg_ref):
        deg = a_ref[...] + b_ref[...] + 1.0
        dis_ref[...] = lax.rsqrt(deg)
        ideg_ref[...] = 1.0 / deg

    return pl.pallas_call(
        body,
        grid=(N_PAD // BLK,),
        in_specs=[
            pl.BlockSpec((BLK, LANES), lambda i: (i, 0)),
            pl.BlockSpec((BLK, LANES), lambda i: (i, 0)),
        ],
        out_specs=[
            pl.BlockSpec((BLK, LANES), lambda i: (i, 0)),
            pl.BlockSpec((BLK, LANES), lambda i: (i, 0)),
        ],
        out_shape=[
            jax.ShapeDtypeStruct((N_PAD, LANES), jnp.float32),
            jax.ShapeDtypeStruct((N_PAD, LANES), jnp.float32),
        ],
    )(dp0, dp1)


def _tc_project(x_pad, wt, dis):
    def body(x_ref, w_ref, s_ref, o_ref):
        y = jnp.dot(x_ref[...], w_ref[...], preferred_element_type=jnp.float32)
        o_ref[...] = y * s_ref[:, 0:1]

    return pl.pallas_call(
        body,
        grid=(N_PAD // BLK,),
        in_specs=[
            pl.BlockSpec((BLK, D), lambda i: (i, 0)),
            pl.BlockSpec((D, C), lambda i: (0, 0)),
            pl.BlockSpec((BLK, LANES), lambda i: (i, 0)),
        ],
        out_specs=pl.BlockSpec((BLK, C), lambda i: (i, 0)),
        out_shape=jax.ShapeDtypeStruct((N_PAD, C), jnp.float32),
    )(x_pad, wt, dis)


def _tc_combine(u, p0, p1, s):
    def body(u_ref, a_ref, b_ref, s_ref, o_ref):
        o_ref[...] = (u_ref[...] + a_ref[...] + b_ref[...]) * s_ref[:, 0:1]

    return pl.pallas_call(
        body,
        grid=(N_PAD // BLK,),
        in_specs=[
            pl.BlockSpec((BLK, C), lambda i: (i, 0)),
            pl.BlockSpec((BLK, C), lambda i: (i, 0)),
            pl.BlockSpec((BLK, C), lambda i: (i, 0)),
            pl.BlockSpec((BLK, LANES), lambda i: (i, 0)),
        ],
        out_specs=pl.BlockSpec((BLK, C), lambda i: (i, 0)),
        out_shape=jax.ShapeDtypeStruct((N_PAD, C), jnp.float32),
    )(u, p0, p1, s)


def _tc_final(u, p0, p1, s, b2d):
    def body(u_ref, a_ref, b_ref, s_ref, bias_ref, o_ref):
        o_ref[...] = (u_ref[...] + a_ref[...] + b_ref[...]) * s_ref[:, 0:1] + bias_ref[...]

    return pl.pallas_call(
        body,
        grid=(N_PAD // BLK,),
        in_specs=[
            pl.BlockSpec((BLK, C), lambda i: (i, 0)),
            pl.BlockSpec((BLK, C), lambda i: (i, 0)),
            pl.BlockSpec((BLK, C), lambda i: (i, 0)),
            pl.BlockSpec((BLK, LANES), lambda i: (i, 0)),
            pl.BlockSpec((1, C), lambda i: (0, 0)),
        ],
        out_specs=pl.BlockSpec((BLK, C), lambda i: (i, 0)),
        out_shape=jax.ShapeDtypeStruct((N_PAD, C), jnp.float32),
    )(u, p0, p1, s, b2d)


@jax.jit
def kernel(x, edge_index, edge_weight, W, b):
    row = edge_index[0].astype(jnp.int32)
    col = edge_index[1].astype(jnp.int32)
    ew = edge_weight.astype(jnp.float32)

    pad = E_PAD - E
    row_r = jnp.pad(row, (0, pad)).reshape(NW, N_CHUNKS, CHUNK)
    col_r = jnp.pad(col, (0, pad)).reshape(NW, N_CHUNKS, CHUNK)
    ew16_r = jnp.broadcast_to(
        jnp.pad(ew, (0, pad)).reshape(NW, N_CHUNKS, CHUNK, 1),
        (NW, N_CHUNKS, CHUNK, LANES)).reshape(NW, N_CHUNKS, CHUNK, LANES)

    x_pad = jnp.pad(x, ((0, N_PAD - N), (0, 0)))
    wt = W.T
    b2d = b.reshape(1, C)

    dp = _deg_kernel(col_r, ew16_r)
    dis, ideg = _tc_deg_norm(dp[0], dp[1])
    u0 = _tc_project(x_pad, wt, dis)
    p = _prop_kernel(u0, row_r, col_r, ew16_r)
    u1 = _tc_combine(u0, p[0], p[1], ideg)
    q = _prop_kernel(u1, row_r, col_r, ew16_r)
    out = _tc_final(u1, q[0], q[1], dis, b2d)
    return out[:N]


# speedup vs baseline: 13.7441x; 1.1229x over previous
import functools

import jax
import jax.numpy as jnp
import numpy as np
from jax import lax
from jax.experimental import pallas as pl
from jax.experimental.pallas import tpu as pltpu
from jax.experimental.pallas import tpu_sc as plsc

N = 10000
E = 320000
D = 128
C = 64

NC = 2
NS = 16
LANES = 16
NW = NC * NS

CHUNK = 128
N_CHUNKS = 80
E_PAD = NW * N_CHUNKS * CHUNK
NBANK = 2
NSTEPS = N_CHUNKS // NBANK
NITER = NSTEPS // 2
DBANK = 8
DSTEPS = N_CHUNKS // DBANK
DITER = DSTEPS // 2
STRIPE = N // NS
SUBS = ((0, CHUNK), (128, CHUNK), (256, CHUNK), (384, CHUNK),
        (512, STRIPE - 4 * CHUNK))


def _sc_mesh():
    return plsc.VectorSubcoreMesh(core_axis_name="c", subcore_axis_name="s")


@functools.partial(
    pl.kernel,
    out_type=jax.ShapeDtypeStruct((NC, N, LANES), jnp.float32),
    mesh=_sc_mesh(),
    scratch_types=[
        pltpu.VMEM((N_CHUNKS, CHUNK), jnp.int32),
        pltpu.VMEM((DBANK, CHUNK, LANES), jnp.float32),
        pltpu.VMEM((DBANK, CHUNK, LANES), jnp.float32),
        pltpu.VMEM((CHUNK, LANES), jnp.float32),
        pltpu.VMEM_SHARED((N, LANES), jnp.float32),
        pltpu.SemaphoreType.DMA,
        pltpu.SemaphoreType.DMA,
        pltpu.SemaphoreType.DMA,
    ],
    compiler_params=pltpu.CompilerParams(use_tc_tiling_on_sc=False),
)
def _deg_kernel(col_hbm, ew16_hbm, out_hbm, col_v, ewa, ewb, zbuf, acc,
                esema, esemb, ssem):
    cid = lax.axis_index("c")
    sid = lax.axis_index("s")
    wid = sid * NC + cid

    pltpu.async_copy(col_hbm.at[wid], col_v, ssem)

    def zrow(i, _):
        zbuf[i, :] = jnp.zeros((LANES,), jnp.float32)
        return 0
    lax.fori_loop(0, CHUNK, zrow, 0)

    for off, ln in SUBS:
        pltpu.sync_copy(zbuf.at[pl.ds(0, ln)],
                        acc.at[pl.ds(sid * STRIPE + off, ln)])
    pltpu.make_async_copy(col_hbm.at[wid], col_v, ssem).wait()
    plsc.subcore_barrier()

    pltpu.async_copy(ew16_hbm.at[wid, pl.ds(0, DBANK)], ewa, esema)
    pltpu.async_copy(ew16_hbm.at[wid, pl.ds(DBANK, DBANK)], ewb, esemb)

    def iter_body(t, _):
        for bank, ew_v, esem in ((0, ewa, esema), (1, ewb, esemb)):
            s = 2 * t + bank
            pltpu.make_async_copy(
                ew16_hbm.at[wid, pl.ds(0, DBANK)], ew_v, esem).wait()
            for b in range(DBANK):
                ch = s * DBANK + b
                pltpu.async_copy(ew_v.at[b], acc.at[col_v.at[ch]], ssem,
                                 add=True)
            for b in range(DBANK):
                pltpu.make_async_copy(
                    ew_v.at[b], acc.at[col_v.at[0]], ssem).wait()

            @pl.when(t < DITER - 1)
            def _():
                pltpu.async_copy(
                    ew16_hbm.at[wid, pl.ds((s + 2) * DBANK, DBANK)], ew_v, esem)
        return 0
    lax.fori_loop(0, DITER, iter_body, 0)

    plsc.subcore_barrier()

    pltpu.sync_copy(acc.at[pl.ds(sid * STRIPE, STRIPE)],
                    out_hbm.at[cid, pl.ds(sid * STRIPE, STRIPE)])


N_SP = N
USTRIPE = N_SP // NS


@functools.partial(
    pl.kernel,
    out_type=jax.ShapeDtypeStruct((NC, N, C), jnp.float32),
    mesh=_sc_mesh(),
    scratch_types=[
        pltpu.VMEM((NBANK, CHUNK), jnp.int32),
        pltpu.VMEM((NBANK, CHUNK), jnp.int32),
        pltpu.VMEM((NBANK, CHUNK), jnp.int32),
        pltpu.VMEM((NBANK, CHUNK), jnp.int32),
        pltpu.VMEM((NBANK, CHUNK, LANES), jnp.float32),
        pltpu.VMEM((NBANK, CHUNK, LANES), jnp.float32),
        pltpu.VMEM((NBANK, CHUNK, C), jnp.float32),
        pltpu.VMEM((NBANK, CHUNK, C), jnp.float32),
        pltpu.VMEM_SHARED((N_SP, C), jnp.float32),
        pltpu.VMEM_SHARED((N_SP, C), jnp.float32),
        pltpu.SemaphoreType.DMA,
        pltpu.SemaphoreType.DMA,
        pltpu.SemaphoreType.DMA,
        pltpu.SemaphoreType.DMA,
        pltpu.SemaphoreType.DMA,
        pltpu.SemaphoreType.DMA,
        pltpu.SemaphoreType.DMA,
    ],
    compiler_params=pltpu.CompilerParams(use_tc_tiling_on_sc=False),
)
def _prop_kernel(u_hbm, row_hbm, col_hbm, ew16_hbm, out_hbm,
                 rowa, rowb, cola, colb, ewa, ewb, bufa, bufb, u_spm, acc,
                 gsema, gsemb, esema, esemb, rsema, rsemb, ssem):
    cid = lax.axis_index("c")
    sid = lax.axis_index("s")
    wid = sid * NC + cid

    ub = sid * USTRIPE
    pltpu.sync_copy(u_hbm.at[pl.ds(ub, USTRIPE)], u_spm.at[pl.ds(ub, USTRIPE)])

    zbuf = bufa.at[0]
    nz = CHUNK * C // LANES

    def zrow(i, _):
        zbuf[i // (C // LANES), pl.ds((i % (C // LANES)) * LANES, LANES)] = (
            jnp.zeros((LANES,), jnp.float32))
        return 0
    lax.fori_loop(0, nz, zrow, 0)

    for off, ln in SUBS:
        pltpu.sync_copy(zbuf.at[pl.ds(0, ln)], acc.at[pl.ds(ub + off, ln)])
    plsc.subcore_barrier()

    pltpu.async_copy(row_hbm.at[wid, pl.ds(0, NBANK)], rowa, rsema)
    pltpu.async_copy(row_hbm.at[wid, pl.ds(NBANK, NBANK)], rowb, rsemb)
    pltpu.async_copy(col_hbm.at[wid, pl.ds(0, NBANK)], cola, esema)
    pltpu.async_copy(ew16_hbm.at[wid, pl.ds(0, NBANK)], ewa, esema)
    pltpu.async_copy(col_hbm.at[wid, pl.ds(NBANK, NBANK)], colb, esemb)
    pltpu.async_copy(ew16_hbm.at[wid, pl.ds(NBANK, NBANK)], ewb, esemb)
    pltpu.make_async_copy(row_hbm.at[wid, pl.ds(0, NBANK)], rowa, rsema).wait()
    pltpu.make_async_copy(row_hbm.at[wid, pl.ds(0, NBANK)], rowb, rsemb).wait()
    for b in range(NBANK):
        pltpu.async_copy(u_spm.at[rowa.at[b]], bufa.at[b], gsema)
    for b in range(NBANK):
        pltpu.async_copy(u_spm.at[rowb.at[b]], bufb.at[b], gsemb)

    def iter_body(t, _):
        for bank, row_v, col_v, ew_v, buf, gsem, esem, rsem in (
                (0, rowa, cola, ewa, bufa, gsema, esema, rsema),
                (1, rowb, colb, ewb, bufb, gsemb, esemb, rsemb)):
            s = 2 * t + bank
            pltpu.make_async_copy(
                col_hbm.at[wid, pl.ds(0, NBANK)], col_v, esem).wait()
            pltpu.make_async_copy(
                ew16_hbm.at[wid, pl.ds(0, NBANK)], ew_v, esem).wait()
            for b in range(NBANK):
                pltpu.make_async_copy(
                    u_spm.at[rowa.at[0]], buf.at[b], gsem).wait()

                def grp(g, _):
                    for t4 in range(4):
                        k = g * 4 + t4
                        w = ew_v[b, k, :]
                        for j in range(C // LANES):
                            sl = pl.ds(j * LANES, LANES)
                            buf[b, k, sl] = buf[b, k, sl] * w
                    return 0
                lax.fori_loop(0, CHUNK // 4, grp, 0)
                pltpu.async_copy(buf.at[b], acc.at[col_v.at[b]], ssem,
                                 add=True)
            for b in range(NBANK):
                pltpu.make_async_copy(
                    buf.at[b], acc.at[col_v.at[0]], ssem).wait()

            @pl.when(t < NITER - 1)
            def _():
                pltpu.async_copy(
                    row_hbm.at[wid, pl.ds((s + 2) * NBANK, NBANK)], row_v, rsem)
                pltpu.async_copy(
                    col_hbm.at[wid, pl.ds((s + 2) * NBANK, NBANK)], col_v, esem)
                pltpu.async_copy(
                    ew16_hbm.at[wid, pl.ds((s + 2) * NBANK, NBANK)], ew_v, esem)
                pltpu.make_async_copy(
                    row_hbm.at[wid, pl.ds(0, NBANK)], row_v, rsem).wait()
                for b in range(NBANK):
                    pltpu.async_copy(u_spm.at[row_v.at[b]], buf.at[b], gsem)
        return 0
    lax.fori_loop(0, NITER, iter_body, 0)

    plsc.subcore_barrier()

    pltpu.sync_copy(acc.at[pl.ds(ub, USTRIPE)],
                    out_hbm.at[cid, pl.ds(ub, USTRIPE)])


BLK = 400


def _tc_ew16(ewp):
    BE = 4096

    def body(w_ref, o_ref):
        o_ref[...] = jnp.broadcast_to(w_ref[...], (BE, LANES))

    return pl.pallas_call(
        body,
        grid=(E_PAD // BE,),
        in_specs=[pl.BlockSpec((BE, 1), lambda i: (i, 0))],
        out_specs=pl.BlockSpec((BE, LANES), lambda i: (i, 0)),
        out_shape=jax.ShapeDtypeStruct((E_PAD, LANES), jnp.float32),
    )(ewp)


def _tc_matmul(x, wt):
    def body(x_ref, w_ref, o_ref):
        o_ref[...] = jnp.dot(x_ref[...], w_ref[...],
                             preferred_element_type=jnp.float32)

    return pl.pallas_call(
        body,
        grid=(N // BLK,),
        in_specs=[
            pl.BlockSpec((BLK, D), lambda i: (i, 0)),
            pl.BlockSpec((D, C), lambda i: (0, 0)),
        ],
        out_specs=pl.BlockSpec((BLK, C), lambda i: (i, 0)),
        out_shape=jax.ShapeDtypeStruct((N, C), jnp.float32),
    )(x, wt)


def _tc_norm(dp0, dp1, y):
    def body(a_ref, b_ref, y_ref, u_ref, dis_ref, ideg_ref):
        deg = a_ref[...] + b_ref[...] + 1.0
        dis = lax.rsqrt(deg)
        dis_ref[...] = dis
        ideg_ref[...] = 1.0 / deg
        u_ref[...] = y_ref[...] * dis[:, 0:1]

    return pl.pallas_call(
        body,
        grid=(N // BLK,),
        in_specs=[
            pl.BlockSpec((BLK, LANES), lambda i: (i, 0)),
            pl.BlockSpec((BLK, LANES), lambda i: (i, 0)),
            pl.BlockSpec((BLK, C), lambda i: (i, 0)),
        ],
        out_specs=[
            pl.BlockSpec((BLK, C), lambda i: (i, 0)),
            pl.BlockSpec((BLK, LANES), lambda i: (i, 0)),
            pl.BlockSpec((BLK, LANES), lambda i: (i, 0)),
        ],
        out_shape=[
            jax.ShapeDtypeStruct((N, C), jnp.float32),
            jax.ShapeDtypeStruct((N, LANES), jnp.float32),
            jax.ShapeDtypeStruct((N, LANES), jnp.float32),
        ],
    )(dp0, dp1, y)


def _tc_combine(u, p0, p1, s):
    def body(u_ref, a_ref, b_ref, s_ref, o_ref):
        o_ref[...] = (u_ref[...] + a_ref[...] + b_ref[...]) * s_ref[:, 0:1]

    return pl.pallas_call(
        body,
        grid=(N // BLK,),
        in_specs=[
            pl.BlockSpec((BLK, C), lambda i: (i, 0)),
            pl.BlockSpec((BLK, C), lambda i: (i, 0)),
            pl.BlockSpec((BLK, C), lambda i: (i, 0)),
            pl.BlockSpec((BLK, LANES), lambda i: (i, 0)),
        ],
        out_specs=pl.BlockSpec((BLK, C), lambda i: (i, 0)),
        out_shape=jax.ShapeDtypeStruct((N, C), jnp.float32),
    )(u, p0, p1, s)


def _tc_final(u, p0, p1, s, b2d):
    def body(u_ref, a_ref, b_ref, s_ref, bias_ref, o_ref):
        o_ref[...] = ((u_ref[...] + a_ref[...] + b_ref[...]) * s_ref[:, 0:1]
                      + bias_ref[...])

    return pl.pallas_call(
        body,
        grid=(N // BLK,),
        in_specs=[
            pl.BlockSpec((BLK, C), lambda i: (i, 0)),
            pl.BlockSpec((BLK, C), lambda i: (i, 0)),
            pl.BlockSpec((BLK, C), lambda i: (i, 0)),
            pl.BlockSpec((BLK, LANES), lambda i: (i, 0)),
            pl.BlockSpec((1, C), lambda i: (0, 0)),
        ],
        out_specs=pl.BlockSpec((BLK, C), lambda i: (i, 0)),
        out_shape=jax.ShapeDtypeStruct((N, C), jnp.float32),
    )(u, p0, p1, s, b2d)


@jax.jit
def kernel(x, edge_index, edge_weight, W, b):
    row = edge_index[0].astype(jnp.int32)
    col = edge_index[1].astype(jnp.int32)
    ew = edge_weight.astype(jnp.float32)

    pad = E_PAD - E
    row_r = jnp.pad(row, (0, pad)).reshape(NW, N_CHUNKS, CHUNK)
    col_r = jnp.pad(col, (0, pad)).reshape(NW, N_CHUNKS, CHUNK)
    ew16_r = _tc_ew16(jnp.pad(ew, (0, pad)).reshape(E_PAD, 1)).reshape(
        NW, N_CHUNKS, CHUNK, LANES)

    wt = W.T
    b2d = b.reshape(1, C)

    dp = _deg_kernel(col_r, ew16_r)
    y = _tc_matmul(x, wt)
    u0, dis, ideg = _tc_norm(dp[0], dp[1], y)
    p = _prop_kernel(u0, row_r, col_r, ew16_r)
    u1 = _tc_combine(u0, p[0], p[1], ideg)
    q = _prop_kernel(u1, row_r, col_r, ew16_r)
    return _tc_final(u1, q[0], q[1], dis, b2d)


# speedup vs baseline: 17.3025x; 1.2589x over previous
import functools

import jax
import jax.numpy as jnp
import numpy as np
from jax import lax
from jax.experimental import pallas as pl
from jax.experimental.pallas import tpu as pltpu
from jax.experimental.pallas import tpu_sc as plsc

N = 10000
E = 320000
D = 128
C = 64

NC = 2
NS = 16
LANES = 16
NW = NC * NS

CHUNK = 128
N_CHUNKS = 80
E_PAD = NW * N_CHUNKS * CHUNK
NBANK = 2
NSTEPS = N_CHUNKS // NBANK
NITER = NSTEPS // 2
DBANK = 8
DSTEPS = N_CHUNKS // DBANK
DITER = DSTEPS // 2
STRIPE = N // NS
SUBS = ((0, CHUNK), (128, CHUNK), (256, CHUNK), (384, CHUNK),
        (512, STRIPE - 4 * CHUNK))


def _sc_mesh():
    return plsc.VectorSubcoreMesh(core_axis_name="c", subcore_axis_name="s")


@functools.partial(
    pl.kernel,
    out_type=jax.ShapeDtypeStruct((NC, N, LANES), jnp.float32),
    mesh=_sc_mesh(),
    scratch_types=[
        pltpu.VMEM((N_CHUNKS, CHUNK), jnp.int32),
        pltpu.VMEM((DBANK, CHUNK, LANES), jnp.float32),
        pltpu.VMEM((DBANK, CHUNK, LANES), jnp.float32),
        pltpu.VMEM((CHUNK, LANES), jnp.float32),
        pltpu.VMEM_SHARED((N, LANES), jnp.float32),
        pltpu.SemaphoreType.DMA,
        pltpu.SemaphoreType.DMA,
        pltpu.SemaphoreType.DMA,
    ],
    compiler_params=pltpu.CompilerParams(use_tc_tiling_on_sc=False),
)
def _deg_kernel(col_hbm, ew16_hbm, out_hbm, col_v, ewa, ewb, zbuf, acc,
                esema, esemb, ssem):
    cid = lax.axis_index("c")
    sid = lax.axis_index("s")
    wid = sid * NC + cid

    pltpu.async_copy(col_hbm.at[wid], col_v, ssem)

    def zrow(i, _):
        zbuf[i, :] = jnp.zeros((LANES,), jnp.float32)
        return 0
    lax.fori_loop(0, CHUNK, zrow, 0)

    for off, ln in SUBS:
        pltpu.sync_copy(zbuf.at[pl.ds(0, ln)],
                        acc.at[pl.ds(sid * STRIPE + off, ln)])
    pltpu.make_async_copy(col_hbm.at[wid], col_v, ssem).wait()
    plsc.subcore_barrier()

    pltpu.async_copy(ew16_hbm.at[wid, pl.ds(0, DBANK)], ewa, esema)
    pltpu.async_copy(ew16_hbm.at[wid, pl.ds(DBANK, DBANK)], ewb, esemb)

    def iter_body(t, _):
        for bank, ew_v, esem in ((0, ewa, esema), (1, ewb, esemb)):
            s = 2 * t + bank
            pltpu.make_async_copy(
                ew16_hbm.at[wid, pl.ds(0, DBANK)], ew_v, esem).wait()
            for b in range(DBANK):
                ch = s * DBANK + b
                pltpu.async_copy(ew_v.at[b], acc.at[col_v.at[ch]], ssem,
                                 add=True)
            for b in range(DBANK):
                pltpu.make_async_copy(
                    ew_v.at[b], acc.at[col_v.at[0]], ssem).wait()

            @pl.when(t < DITER - 1)
            def _():
                pltpu.async_copy(
                    ew16_hbm.at[wid, pl.ds((s + 2) * DBANK, DBANK)], ew_v, esem)
        return 0
    lax.fori_loop(0, DITER, iter_body, 0)

    plsc.subcore_barrier()

    pltpu.sync_copy(acc.at[pl.ds(sid * STRIPE, STRIPE)],
                    out_hbm.at[cid, pl.ds(sid * STRIPE, STRIPE)])


N_SP = N
USTRIPE = N_SP // NS


@functools.partial(
    pl.kernel,
    out_type=jax.ShapeDtypeStruct((NC, N, C), jnp.float32),
    mesh=_sc_mesh(),
    scratch_types=[
        pltpu.VMEM((NBANK, CHUNK), jnp.int32),
        pltpu.VMEM((NBANK, CHUNK), jnp.int32),
        pltpu.VMEM((NBANK, CHUNK), jnp.int32),
        pltpu.VMEM((NBANK, CHUNK), jnp.int32),
        pltpu.VMEM((NBANK, CHUNK, LANES), jnp.float32),
        pltpu.VMEM((NBANK, CHUNK, LANES), jnp.float32),
        pltpu.VMEM((NBANK, CHUNK, C), jnp.float32),
        pltpu.VMEM((NBANK, CHUNK, C), jnp.float32),
        pltpu.VMEM_SHARED((N_SP, C), jnp.float32),
        pltpu.VMEM_SHARED((N_SP, C), jnp.float32),
        pltpu.SemaphoreType.DMA,
        pltpu.SemaphoreType.DMA,
        pltpu.SemaphoreType.DMA,
        pltpu.SemaphoreType.DMA,
        pltpu.SemaphoreType.DMA,
        pltpu.SemaphoreType.DMA,
        pltpu.SemaphoreType.DMA,
    ],
    compiler_params=pltpu.CompilerParams(use_tc_tiling_on_sc=False),
)
def _prop_kernel(u_hbm, row_hbm, col_hbm, ew16_hbm, out_hbm,
                 rowa, rowb, cola, colb, ewa, ewb, bufa, bufb, u_spm, acc,
                 gsema, gsemb, esema, esemb, rsema, rsemb, ssem):
    cid = lax.axis_index("c")
    sid = lax.axis_index("s")
    wid = sid * NC + cid

    ub = sid * USTRIPE
    pltpu.sync_copy(u_hbm.at[pl.ds(ub, USTRIPE)], u_spm.at[pl.ds(ub, USTRIPE)])

    zbuf = bufa.at[0]
    nz = CHUNK * C // LANES

    def zrow(i, _):
        zbuf[i // (C // LANES), pl.ds((i % (C // LANES)) * LANES, LANES)] = (
            jnp.zeros((LANES,), jnp.float32))
        return 0
    lax.fori_loop(0, nz, zrow, 0)

    for off, ln in SUBS:
        pltpu.sync_copy(zbuf.at[pl.ds(0, ln)], acc.at[pl.ds(ub + off, ln)])
    plsc.subcore_barrier()

    pltpu.async_copy(row_hbm.at[wid, pl.ds(0, NBANK)], rowa, rsema)
    pltpu.async_copy(row_hbm.at[wid, pl.ds(NBANK, NBANK)], rowb, rsemb)
    pltpu.async_copy(col_hbm.at[wid, pl.ds(0, NBANK)], cola, esema)
    pltpu.async_copy(ew16_hbm.at[wid, pl.ds(0, NBANK)], ewa, esema)
    pltpu.async_copy(col_hbm.at[wid, pl.ds(NBANK, NBANK)], colb, esemb)
    pltpu.async_copy(ew16_hbm.at[wid, pl.ds(NBANK, NBANK)], ewb, esemb)
    pltpu.make_async_copy(row_hbm.at[wid, pl.ds(0, NBANK)], rowa, rsema).wait()
    pltpu.make_async_copy(row_hbm.at[wid, pl.ds(0, NBANK)], rowb, rsemb).wait()
    for b in range(NBANK):
        pltpu.async_copy(u_spm.at[rowa.at[b]], bufa.at[b], gsema)
    for b in range(NBANK):
        pltpu.async_copy(u_spm.at[rowb.at[b]], bufb.at[b], gsemb)

    def iter_body(t, _):
        for bank, row_v, col_v, ew_v, buf, gsem, esem, rsem in (
                (0, rowa, cola, ewa, bufa, gsema, esema, rsema),
                (1, rowb, colb, ewb, bufb, gsemb, esemb, rsemb)):
            s = 2 * t + bank
            pltpu.make_async_copy(
                col_hbm.at[wid, pl.ds(0, NBANK)], col_v, esem).wait()
            pltpu.make_async_copy(
                ew16_hbm.at[wid, pl.ds(0, NBANK)], ew_v, esem).wait()
            for b in range(NBANK):
                pltpu.make_async_copy(
                    u_spm.at[rowa.at[0]], buf.at[b], gsem).wait()

                def grp(g, _):
                    for t4 in range(4):
                        k = g * 4 + t4
                        w = ew_v[b, k, :]
                        for j in range(C // LANES):
                            sl = pl.ds(j * LANES, LANES)
                            buf[b, k, sl] = buf[b, k, sl] * w
                    return 0
                lax.fori_loop(0, CHUNK // 4, grp, 0)
                pltpu.async_copy(buf.at[b], acc.at[col_v.at[b]], ssem,
                                 add=True)
            for b in range(NBANK):
                pltpu.make_async_copy(
                    buf.at[b], acc.at[col_v.at[0]], ssem).wait()

            @pl.when(t < NITER - 1)
            def _():
                pltpu.async_copy(
                    row_hbm.at[wid, pl.ds((s + 2) * NBANK, NBANK)], row_v, rsem)
                pltpu.async_copy(
                    col_hbm.at[wid, pl.ds((s + 2) * NBANK, NBANK)], col_v, esem)
                pltpu.async_copy(
                    ew16_hbm.at[wid, pl.ds((s + 2) * NBANK, NBANK)], ew_v, esem)
                pltpu.make_async_copy(
                    row_hbm.at[wid, pl.ds(0, NBANK)], row_v, rsem).wait()
                for b in range(NBANK):
                    pltpu.async_copy(u_spm.at[row_v.at[b]], buf.at[b], gsem)
        return 0
    lax.fori_loop(0, NITER, iter_body, 0)

    plsc.subcore_barrier()

    pltpu.sync_copy(acc.at[pl.ds(ub, USTRIPE)],
                    out_hbm.at[cid, pl.ds(ub, USTRIPE)])


BLK = 400


def _tc_matmul(x, wt):
    def body(x_ref, w_ref, o_ref):
        o_ref[...] = jnp.dot(x_ref[...], w_ref[...],
                             preferred_element_type=jnp.float32)

    return pl.pallas_call(
        body,
        grid=(N // BLK,),
        in_specs=[
            pl.BlockSpec((BLK, D), lambda i: (i, 0)),
            pl.BlockSpec((D, C), lambda i: (0, 0)),
        ],
        out_specs=pl.BlockSpec((BLK, C), lambda i: (i, 0)),
        out_shape=jax.ShapeDtypeStruct((N, C), jnp.float32),
    )(x, wt)


def _tc_norm(dp0, dp1, y):
    def body(a_ref, b_ref, y_ref, u_ref, dis_ref, ideg_ref):
        deg = a_ref[...] + b_ref[...] + 1.0
        dis = lax.rsqrt(deg)
        dis_ref[...] = dis
        ideg_ref[...] = 1.0 / deg
        u_ref[...] = y_ref[...] * dis[:, 0:1]

    return pl.pallas_call(
        body,
        grid=(N // BLK,),
        in_specs=[
            pl.BlockSpec((BLK, LANES), lambda i: (i, 0)),
            pl.BlockSpec((BLK, LANES), lambda i: (i, 0)),
            pl.BlockSpec((BLK, C), lambda i: (i, 0)),
        ],
        out_specs=[
            pl.BlockSpec((BLK, C), lambda i: (i, 0)),
            pl.BlockSpec((BLK, LANES), lambda i: (i, 0)),
            pl.BlockSpec((BLK, LANES), lambda i: (i, 0)),
        ],
        out_shape=[
            jax.ShapeDtypeStruct((N, C), jnp.float32),
            jax.ShapeDtypeStruct((N, LANES), jnp.float32),
            jax.ShapeDtypeStruct((N, LANES), jnp.float32),
        ],
    )(dp0, dp1, y)


def _tc_combine(u, p0, p1, s):
    def body(u_ref, a_ref, b_ref, s_ref, o_ref):
        o_ref[...] = (u_ref[...] + a_ref[...] + b_ref[...]) * s_ref[:, 0:1]

    return pl.pallas_call(
        body,
        grid=(N // BLK,),
        in_specs=[
            pl.BlockSpec((BLK, C), lambda i: (i, 0)),
            pl.BlockSpec((BLK, C), lambda i: (i, 0)),
            pl.BlockSpec((BLK, C), lambda i: (i, 0)),
            pl.BlockSpec((BLK, LANES), lambda i: (i, 0)),
        ],
        out_specs=pl.BlockSpec((BLK, C), lambda i: (i, 0)),
        out_shape=jax.ShapeDtypeStruct((N, C), jnp.float32),
    )(u, p0, p1, s)


def _tc_final(u, p0, p1, s, b2d):
    def body(u_ref, a_ref, b_ref, s_ref, bias_ref, o_ref):
        o_ref[...] = ((u_ref[...] + a_ref[...] + b_ref[...]) * s_ref[:, 0:1]
                      + bias_ref[...])

    return pl.pallas_call(
        body,
        grid=(N // BLK,),
        in_specs=[
            pl.BlockSpec((BLK, C), lambda i: (i, 0)),
            pl.BlockSpec((BLK, C), lambda i: (i, 0)),
            pl.BlockSpec((BLK, C), lambda i: (i, 0)),
            pl.BlockSpec((BLK, LANES), lambda i: (i, 0)),
            pl.BlockSpec((1, C), lambda i: (0, 0)),
        ],
        out_specs=pl.BlockSpec((BLK, C), lambda i: (i, 0)),
        out_shape=jax.ShapeDtypeStruct((N, C), jnp.float32),
    )(u, p0, p1, s, b2d)


@jax.jit
def kernel(x, edge_index, edge_weight, W, b):
    row = edge_index[0].astype(jnp.int32)
    col = edge_index[1].astype(jnp.int32)
    ew = edge_weight.astype(jnp.float32)

    pad = E_PAD - E
    row_r = jnp.pad(row, (0, pad)).reshape(NW, N_CHUNKS, CHUNK)
    col_r = jnp.pad(col, (0, pad)).reshape(NW, N_CHUNKS, CHUNK)
    ew16_r = jnp.broadcast_to(
        jnp.pad(ew, (0, pad)).reshape(NW, N_CHUNKS, CHUNK, 1),
        (NW, N_CHUNKS, CHUNK, LANES)).reshape(NW, N_CHUNKS, CHUNK, LANES)

    wt = W.T
    b2d = b.reshape(1, C)

    dp = _deg_kernel(col_r, ew16_r)
    y = _tc_matmul(x, wt)
    u0, dis, ideg = _tc_norm(dp[0], dp[1], y)
    p = _prop_kernel(u0, row_r, col_r, ew16_r)
    u1 = _tc_combine(u0, p[0], p[1], ideg)
    q = _prop_kernel(u1, row_r, col_r, ew16_r)
    return _tc_final(u1, q[0], q[1], dis, b2d)
